# Initial kernel scaffold; baseline (speedup 1.0000x reference)
import jax
import jax.numpy as jnp
from jax.experimental import pallas as pl


def kernel(data_or_x, batch, atoms):
    raise NotImplementedError("write your pallas kernel here")


# speedup vs baseline: 75.6440x; 75.6440x over previous
import functools

import jax
import jax.numpy as jnp
from jax.experimental import pallas as pl
from jax.experimental.pallas import tpu as pltpu

_NUM_GRAPHS = 64
_BLK = 512


def _mmd_body(nt, num_graphs, kk, cc, gamma,
              tb_ref, xA_ref, xB_ref, bl_ref, st_ref, en_ref, a_ref, aT_ref,
              out_ref, sxx_ref, sxy_ref, cnt_ref):
    i = pl.program_id(0)
    blk = xA_ref.shape[2]

    @pl.when(i == 0)
    def _init():
        sxx_ref[...] = jnp.zeros_like(sxx_ref)
        sxy_ref[...] = jnp.zeros_like(sxy_ref)
        cnt_ref[...] = jnp.zeros_like(cnt_ref)

    xr = xB_ref[i]
    br = bl_ref[i]
    st = st_ref[i]
    en = en_ref[i]

    t0 = tb_ref[i, 0]
    t1 = tb_ref[i, 1]

    def inner(t, rs):
        xc = xA_ref[t]
        nd = jax.lax.dot_general(
            xc, xr, (((0,), (0,)), ((), ())),
            preferred_element_type=jnp.float32)
        e = jnp.exp(nd)
        cidx = t * blk + jax.lax.broadcasted_iota(jnp.int32, (blk, 1), 0)
        m = (cidx >= st) & (cidx < en)
        e = jnp.where(m, e, 0.0)
        return rs + jnp.sum(e, axis=0, keepdims=True)

    rs = jax.lax.fori_loop(t0, t1 + 1, inner,
                           jnp.zeros((1, blk), jnp.float32))

    gid = jax.lax.broadcasted_iota(jnp.int32, (num_graphs, 1), 0)
    oh = (gid == br).astype(jnp.float32)
    sxx_ref[...] += jnp.sum(oh * rs, axis=1, keepdims=True)
    cnt_ref[...] += jnp.sum(oh, axis=1, keepdims=True)

    a = a_ref[...]
    a2 = jnp.sum(a * a, axis=1, keepdims=True)
    nd_feat = xA_ref.shape[1] - 2
    x_rows = xr[0:nd_feat, :]
    gn2 = xr[nd_feat:nd_feat + 1, :]
    ndxy = jax.lax.dot_general(
        a, x_rows, (((1,), (0,)), ((), ())),
        preferred_element_type=jnp.float32) - gamma * a2 + gn2
    exy = jnp.exp(ndxy)
    rid = jax.lax.broadcasted_iota(jnp.int32, (cc * kk, 1), 0) // kk
    cols = []
    for c in range(cc):
        sc = jnp.sum(jnp.where(rid == c, exy, 0.0), axis=0, keepdims=True)
        cols.append(jnp.sum(oh * sc, axis=1, keepdims=True))
    sxy_ref[...] += jnp.concatenate(cols, axis=1)

    @pl.when(i == nt - 1)
    def _fin():
        cnt = cnt_ref[...]
        kxx = sxx_ref[...] / (cnt * cnt)
        aT = aT_ref[...]
        a2l = jnp.sum(aT * aT, axis=0, keepdims=True)
        ndyy = 2.0 * gamma * jax.lax.dot_general(
            a, aT, (((1,), (0,)), ((), ())),
            preferred_element_type=jnp.float32) - gamma * (a2 + a2l)
        eyy = jnp.exp(ndyy)
        cidy = jax.lax.broadcasted_iota(jnp.int32, (1, cc * kk), 1) // kk
        outs = []
        for c in range(cc):
            msk = (rid == c) & (cidy == c)
            kyy_c = jnp.sum(jnp.where(msk, eyy, 0.0)) / (kk * kk)
            kxy_c = sxy_ref[:, c:c + 1] / (cnt * kk)
            outs.append(kxx + kyy_c - 2.0 * kxy_c)
        out_ref[...] = jnp.concatenate(outs, axis=1)


def kernel(data_or_x, batch, atoms):
    x = data_or_x
    n, d = x.shape
    cc, kk, _ = atoms.shape
    g = _NUM_GRAPHS
    gamma = 1.0 / d
    blk = _BLK

    nt = -(-n // blk)
    npad = nt * blk
    pad = npad - n

    xp = jnp.pad(x.astype(jnp.float32), ((0, pad), (0, 0)))
    bp = jnp.pad(batch.astype(jnp.int32), (0, pad), constant_values=g)

    n2 = jnp.sum(xp * xp, axis=1)
    offsets = jnp.searchsorted(
        bp, jnp.arange(g + 2, dtype=jnp.int32)).astype(jnp.int32)
    seg_st = offsets[bp]
    seg_en = offsets[bp + 1]

    xb = xp.reshape(nt, blk, d).transpose(0, 2, 1)
    n2b = n2.reshape(nt, 1, blk)
    ones = jnp.ones((nt, 1, blk), jnp.float32)
    xA = jnp.concatenate([xb, ones, -gamma * n2b], axis=1)
    xB = jnp.concatenate([2.0 * gamma * xb, -gamma * n2b, ones], axis=1)

    bl3 = bp.reshape(nt, 1, blk)
    st3 = seg_st.reshape(nt, 1, blk)
    en3 = seg_en.reshape(nt, 1, blk)

    r0 = jnp.arange(nt, dtype=jnp.int32) * blk
    g_lo = bp[r0]
    g_hi = bp[r0 + blk - 1]
    c_lo = offsets[g_lo]
    c_hi = offsets[g_hi + 1]
    t_lo = c_lo // blk
    t_hi = jnp.maximum(c_hi - 1, c_lo) // blk
    tb = jnp.stack([t_lo, t_hi], axis=1).astype(jnp.int32)

    af = atoms.reshape(cc * kk, d).astype(jnp.float32)
    afT = af.T

    grid_spec = pltpu.PrefetchScalarGridSpec(
        num_scalar_prefetch=1,
        grid=(nt,),
        in_specs=[
            pl.BlockSpec(xA.shape, lambda i, tb_: (0, 0, 0)),
            pl.BlockSpec(xB.shape, lambda i, tb_: (0, 0, 0)),
            pl.BlockSpec(bl3.shape, lambda i, tb_: (0, 0, 0)),
            pl.BlockSpec(st3.shape, lambda i, tb_: (0, 0, 0)),
            pl.BlockSpec(en3.shape, lambda i, tb_: (0, 0, 0)),
            pl.BlockSpec(af.shape, lambda i, tb_: (0, 0)),
            pl.BlockSpec(afT.shape, lambda i, tb_: (0, 0)),
        ],
        out_specs=pl.BlockSpec((g, cc), lambda i, tb_: (0, 0)),
        scratch_shapes=[
            pltpu.VMEM((g, 1), jnp.float32),
            pltpu.VMEM((g, cc), jnp.float32),
            pltpu.VMEM((g, 1), jnp.float32),
        ],
    )
    body = functools.partial(_mmd_body, nt, g, kk, cc, gamma)
    out = pl.pallas_call(
        body,
        grid_spec=grid_spec,
        out_shape=jax.ShapeDtypeStruct((g, cc), jnp.float32),
        compiler_params=pltpu.CompilerParams(
            dimension_semantics=("arbitrary",)),
    )(tb, xA, xB, bl3, st3, en3, af, afT)
    return out


# speedup vs baseline: 498.0389x; 6.5840x over previous
import functools

import jax
import jax.numpy as jnp
from jax.experimental import pallas as pl
from jax.experimental.pallas import tpu as pltpu

_NUM_GRAPHS = 64
_BLK = 512
_NFEAT = 24
_BIG = 128.0


def _mmd_body(nt, num_graphs, kk, cc, gamma,
              tb_ref, xA_ref, xB_ref, bl_ref, a_ref, aT_ref,
              out_ref, sxx_ref, sxy_ref, cnt_ref):
    i = pl.program_id(0)
    blk = xA_ref.shape[2]

    @pl.when(i == 0)
    def _init():
        sxx_ref[...] = jnp.zeros_like(sxx_ref)
        sxy_ref[...] = jnp.zeros_like(sxy_ref)
        cnt_ref[...] = jnp.zeros_like(cnt_ref)

    xr = xB_ref[i]
    br = bl_ref[i]

    gid = jax.lax.broadcasted_iota(jnp.int32, (num_graphs, 1), 0)
    oh = (gid == br).astype(jnp.float32)
    rcat = jnp.concatenate([xr, oh], axis=0)

    t0 = tb_ref[i, 0]
    t1 = tb_ref[i, 1]

    def inner(t, rs):
        xc = xA_ref[t]
        bc = bl_ref[t]
        ohc = jnp.where(gid == bc, _BIG, 0.0)
        lcat = jnp.concatenate([xc, ohc], axis=0)
        nd = jax.lax.dot_general(
            lcat, rcat, (((0,), (0,)), ((), ())),
            preferred_element_type=jnp.float32)
        e = jnp.exp(nd)
        return rs + jnp.sum(e, axis=0, keepdims=True)

    rs = jax.lax.fori_loop(t0, t1 + 1, inner,
                           jnp.zeros((1, blk), jnp.float32))

    sxx_ref[...] += jnp.sum(oh * rs, axis=1, keepdims=True)
    cnt_ref[...] += jnp.sum(oh, axis=1, keepdims=True)

    a = a_ref[...]
    a2 = jnp.sum(a * a, axis=1, keepdims=True)
    nd_feat = a_ref.shape[1]
    x_rows = xr[0:nd_feat, :]
    gn2 = xr[nd_feat:nd_feat + 1, :]
    ndxy = jax.lax.dot_general(
        a, x_rows, (((1,), (0,)), ((), ())),
        preferred_element_type=jnp.float32) - gamma * a2 + gn2
    exy = jnp.exp(ndxy)
    rid = jax.lax.broadcasted_iota(jnp.int32, (cc * kk, 1), 0) // kk
    cols = []
    for c in range(cc):
        sc = jnp.sum(jnp.where(rid == c, exy, 0.0), axis=0, keepdims=True)
        cols.append(jnp.sum(oh * sc, axis=1, keepdims=True))
    sxy_ref[...] += jnp.concatenate(cols, axis=1)

    @pl.when(i == nt - 1)
    def _fin():
        cnt = cnt_ref[...]
        kxx = sxx_ref[...] / (cnt * cnt)
        aT = aT_ref[...]
        a2l = jnp.sum(aT * aT, axis=0, keepdims=True)
        ndyy = 2.0 * gamma * jax.lax.dot_general(
            a, aT, (((1,), (0,)), ((), ())),
            preferred_element_type=jnp.float32) - gamma * (a2 + a2l)
        eyy = jnp.exp(ndyy)
        cidy = jax.lax.broadcasted_iota(jnp.int32, (1, cc * kk), 1) // kk
        outs = []
        for c in range(cc):
            msk = (rid == c) & (cidy == c)
            kyy_c = jnp.sum(jnp.where(msk, eyy, 0.0)) / (kk * kk)
            kxy_c = sxy_ref[:, c:c + 1] / (cnt * kk)
            outs.append(kxx + kyy_c - 2.0 * kxy_c)
        out_ref[...] = jnp.concatenate(outs, axis=1)


def kernel(data_or_x, batch, atoms):
    x = data_or_x
    n, d = x.shape
    cc, kk, _ = atoms.shape
    g = _NUM_GRAPHS
    gamma = 1.0 / d
    blk = _BLK

    nt = -(-n // blk)
    npad = nt * blk
    pad = npad - n

    xp = jnp.pad(x.astype(jnp.float32), ((0, pad), (0, 0)))
    bp = jnp.pad(batch.astype(jnp.int32), (0, pad), constant_values=g)

    n2 = jnp.sum(xp * xp, axis=1)
    offsets = jnp.searchsorted(
        bp, jnp.arange(g + 2, dtype=jnp.int32)).astype(jnp.int32)

    xb = xp.reshape(nt, blk, d).transpose(0, 2, 1)
    n2b = n2.reshape(nt, 1, blk)
    ones = jnp.ones((nt, 1, blk), jnp.float32)
    zeros = jnp.zeros((nt, _NFEAT - d - 3, blk), jnp.float32)
    xA = jnp.concatenate([xb, ones, -gamma * n2b, ones, zeros], axis=1)
    xB = jnp.concatenate(
        [2.0 * gamma * xb, -gamma * n2b, ones, -_BIG * ones, zeros], axis=1)

    bl3 = bp.reshape(nt, 1, blk)

    r0 = jnp.arange(nt, dtype=jnp.int32) * blk
    g_lo = bp[r0]
    g_hi = bp[r0 + blk - 1]
    c_lo = offsets[g_lo]
    c_hi = offsets[g_hi + 1]
    t_lo = c_lo // blk
    t_hi = jnp.maximum(c_hi - 1, c_lo) // blk
    tb = jnp.stack([t_lo, t_hi], axis=1).astype(jnp.int32)

    af = atoms.reshape(cc * kk, d).astype(jnp.float32)
    afT = af.T

    grid_spec = pltpu.PrefetchScalarGridSpec(
        num_scalar_prefetch=1,
        grid=(nt,),
        in_specs=[
            pl.BlockSpec(xA.shape, lambda i, tb_: (0, 0, 0)),
            pl.BlockSpec(xB.shape, lambda i, tb_: (0, 0, 0)),
            pl.BlockSpec(bl3.shape, lambda i, tb_: (0, 0, 0)),
            pl.BlockSpec(af.shape, lambda i, tb_: (0, 0)),
            pl.BlockSpec(afT.shape, lambda i, tb_: (0, 0)),
        ],
        out_specs=pl.BlockSpec((g, cc), lambda i, tb_: (0, 0)),
        scratch_shapes=[
            pltpu.VMEM((g, 1), jnp.float32),
            pltpu.VMEM((g, cc), jnp.float32),
            pltpu.VMEM((g, 1), jnp.float32),
        ],
    )
    body = functools.partial(_mmd_body, nt, g, kk, cc, gamma)
    out = pl.pallas_call(
        body,
        grid_spec=grid_spec,
        out_shape=jax.ShapeDtypeStruct((g, cc), jnp.float32),
        compiler_params=pltpu.CompilerParams(
            dimension_semantics=("arbitrary",)),
    )(tb, xA, xB, bl3, af, afT)
    return out
